# Initial kernel scaffold; baseline (speedup 1.0000x reference)
import jax
import jax.numpy as jnp
from jax.experimental import pallas as pl


def kernel(x, edge_index, edge_attr, batch, charge, params):
    raise NotImplementedError("write your pallas kernel here")


# speedup vs baseline: 2.4313x; 2.4313x over previous
import functools

import jax
import jax.numpy as jnp
from jax import lax
from jax.experimental import pallas as pl
from jax.experimental.pallas import tpu as pltpu
from jax.experimental.pallas import tpu_sc as plsc

N = 10000
E = 320000
AF = 128
BF = 16
D = 64
H = 128
G = 64

NC = 2
NS = 16
NW = NC * NS
PER_W = E // NW
CH = 80
NCH = PER_W // CH
ROWS_PER_SUB = N // NS

EB = 2000
NB = 2000

_F32 = jnp.float32


def _sp(x):
    return jnp.maximum(x, 0.0) + jnp.log1p(jnp.exp(-jnp.abs(x)))


def _mesh():
    return plsc.VectorSubcoreMesh(core_axis_name="c", subcore_axis_name="s")


def _gather2(t1, t2, row3, col3):
    @functools.partial(
        pl.kernel,
        out_type=[jax.ShapeDtypeStruct((E, 2 * D), _F32),
                  jax.ShapeDtypeStruct((E, 2 * D), _F32)],
        mesh=_mesh(),
        scratch_types=[
            pltpu.VMEM((NCH, CH), jnp.int32),
            pltpu.VMEM((NCH, CH), jnp.int32),
            pltpu.VMEM((CH, 2 * D), _F32),
            pltpu.VMEM((CH, 2 * D), _F32),
            pltpu.SemaphoreType.DMA,
            pltpu.SemaphoreType.DMA,
        ],
    )
    def k(t1_hbm, t2_hbm, row_hbm, col_hbm, r_hbm, c_hbm, ri, ci, rb, cb,
          sem1, sem2):
        wid = lax.axis_index("s") * NC + lax.axis_index("c")
        base = wid * PER_W
        pltpu.sync_copy(row_hbm.at[wid], ri)
        pltpu.sync_copy(col_hbm.at[wid], ci)

        @pl.loop(0, NCH)
        def _(j):
            c1 = pltpu.async_copy(t1_hbm.at[ri.at[j]], rb, sem1)
            c2 = pltpu.async_copy(t2_hbm.at[ci.at[j]], cb, sem2)
            c1.wait()
            c2.wait()
            off = base + j * CH
            pltpu.sync_copy(rb, r_hbm.at[pl.ds(off, CH)])
            pltpu.sync_copy(cb, c_hbm.at[pl.ds(off, CH)])

    return k(t1, t2, row3, col3)


def _node_tables(h2, cp):
    w1, b1 = cp['eu1']
    wn1, bn1 = cp['nu1']
    w1a, w1b = w1[0:D], w1[D:2 * D]
    wn1a = wn1[0:D]

    def body(h_ref, w1a_ref, w1b_ref, wn1a_ref, b1_ref, bn1_ref,
             t1_ref, t2_ref):
        h = h_ref[:, 0:D]
        t1_ref[:, 0:D] = (jnp.dot(h, w1a_ref[...],
                                  preferred_element_type=_F32) + b1_ref[...])
        t1_ref[:, D:2 * D] = (jnp.dot(h, wn1a_ref[...],
                                      preferred_element_type=_F32)
                              + bn1_ref[...])
        t2_ref[:, 0:D] = jnp.dot(h, w1b_ref[...], preferred_element_type=_F32)
        t2_ref[:, D:2 * D] = jnp.zeros((NB, D), _F32)

    hspec = pl.BlockSpec((NB, 2 * D), lambda i: (i, 0))
    wspec = pl.BlockSpec((D, D), lambda i: (0, 0))
    bspec = pl.BlockSpec((1, D), lambda i: (0, 0))
    return pl.pallas_call(
        body,
        grid=(N // NB,),
        in_specs=[hspec, wspec, wspec, wspec, bspec, bspec],
        out_specs=[hspec, hspec],
        out_shape=[jax.ShapeDtypeStruct((N, 2 * D), _F32),
                   jax.ShapeDtypeStruct((N, 2 * D), _F32)],
    )(h2, w1a, w1b, wn1a, b1.reshape(1, D), bn1.reshape(1, D))


def _scatter_add(msg, col3):
    @functools.partial(
        pl.kernel,
        out_type=jax.ShapeDtypeStruct((NC, N, D), _F32),
        mesh=_mesh(),
        scratch_types=[
            pltpu.VMEM((NCH, CH), jnp.int32),
            pltpu.VMEM((CH, D), _F32),
            pltpu.VMEM((104, D), _F32),
            pltpu.VMEM_SHARED((N, D), _F32),
        ],
    )
    def k(msg_hbm, col_hbm, out_hbm, ci, mb, zb, hsh):
        c = lax.axis_index("c")
        s = lax.axis_index("s")

        @pl.loop(0, 104)
        def _(r):
            @pl.loop(0, D, step=16)
            def _(q):
                zb[r, pl.ds(q, 16)] = jnp.zeros((16,), _F32)

        @pl.loop(0, 6)
        def _(t):
            pltpu.sync_copy(zb, hsh.at[pl.ds(s * 624 + t * 104, 104)])

        @pl.when(s == 0)
        def _():
            pltpu.sync_copy(zb.at[pl.ds(0, 16)], hsh.at[pl.ds(9984, 16)])

        plsc.subcore_barrier()

        wid = s * NC + c
        pltpu.sync_copy(col_hbm.at[wid], ci)

        @pl.loop(0, NCH)
        def _(j):
            off = wid * PER_W + j * CH
            pltpu.sync_copy(msg_hbm.at[pl.ds(off, CH)], mb)
            pltpu.sync_copy(mb, hsh.at[ci.at[j]], add=True)

        plsc.subcore_barrier()
        pltpu.sync_copy(hsh.at[pl.ds(s * 624, 624)],
                        out_hbm.at[c, pl.ds(s * 624, 624)])

        @pl.when(s == 0)
        def _():
            pltpu.sync_copy(hsh.at[pl.ds(9984, 16)],
                            out_hbm.at[c, pl.ds(9984, 16)])

    return k(msg, col3)


def _embed(x, w, b, bm):
    m, fin = x.shape
    fout = w.shape[1]

    def body(x_ref, w_ref, b_ref, o_ref):
        o_ref[...] = (jnp.dot(x_ref[...], w_ref[...],
                              preferred_element_type=_F32) + b_ref[...])

    return pl.pallas_call(
        body,
        grid=(m // bm,),
        in_specs=[pl.BlockSpec((bm, fin), lambda i: (i, 0)),
                  pl.BlockSpec((fin, fout), lambda i: (0, 0)),
                  pl.BlockSpec((1, fout), lambda i: (0, 0))],
        out_specs=pl.BlockSpec((bm, fout), lambda i: (i, 0)),
        out_shape=jax.ShapeDtypeStruct((m, fout), _F32),
    )(x, w, b.reshape(1, fout))


def _edge_mlp(r, c, e, cp):
    w1 = cp['eu1'][0]
    w2, b2 = cp['eu2']
    wn1 = cp['nu1'][0]
    wn2, bn2 = cp['nu2']
    w1c = w1[2 * D:3 * D]
    wn1b = wn1[D:2 * D]

    def body(r_ref, c_ref, e_ref, w1c_ref, w2_ref, b2_ref, wn1b_ref,
             wn2_ref, bn2_ref, en_ref, msg_ref):
        t = (r_ref[:, 0:D] + c_ref[:, 0:D]
             + jnp.dot(e_ref[...], w1c_ref[...], preferred_element_type=_F32))
        t = _sp(t)
        en = jnp.dot(t, w2_ref[...], preferred_element_type=_F32) + b2_ref[...]
        en_ref[...] = en
        u = _sp(r_ref[:, D:2 * D]
                + jnp.dot(en, wn1b_ref[...], preferred_element_type=_F32))
        msg_ref[...] = (jnp.dot(u, wn2_ref[...], preferred_element_type=_F32)
                        + bn2_ref[...])

    wspec = pl.BlockSpec((D, D), lambda i: (0, 0))
    bspec = pl.BlockSpec((1, D), lambda i: (0, 0))
    espec = pl.BlockSpec((EB, D), lambda i: (i, 0))
    gspec = pl.BlockSpec((EB, 2 * D), lambda i: (i, 0))
    return pl.pallas_call(
        body,
        grid=(E // EB,),
        in_specs=[gspec, gspec, espec,
                  wspec, wspec, bspec, wspec, wspec, bspec],
        out_specs=[espec, espec],
        out_shape=[jax.ShapeDtypeStruct((E, D), _F32),
                   jax.ShapeDtypeStruct((E, D), _F32)],
    )(r, c, e, w1c, w2, b2.reshape(1, D), wn1b, wn2, bn2.reshape(1, D))


def _bn_residual(parts, h2, g, b):
    def body(p_ref, h_ref, g_ref, b_ref, o_ref):
        s = p_ref[0] + p_ref[1]
        mu = jnp.mean(s, axis=0, keepdims=True)
        var = jnp.mean((s - mu) * (s - mu), axis=0, keepdims=True)
        hb = (s - mu) / jnp.sqrt(var + 1e-5) * g_ref[...] + b_ref[...]
        hold = h_ref[:, 0:D]
        o_ref[:, 0:D] = _sp(hb) + hold
        o_ref[:, D:2 * D] = jnp.zeros((N, D), _F32)

    return pl.pallas_call(
        body,
        grid=(1,),
        in_specs=[pl.BlockSpec((NC, N, D), lambda i: (0, 0, 0)),
                  pl.BlockSpec((N, 2 * D), lambda i: (0, 0)),
                  pl.BlockSpec((1, D), lambda i: (0, 0)),
                  pl.BlockSpec((1, D), lambda i: (0, 0))],
        out_specs=pl.BlockSpec((N, 2 * D), lambda i: (0, 0)),
        out_shape=jax.ShapeDtypeStruct((N, 2 * D), _F32),
    )(parts, h2, g.reshape(1, D), b.reshape(1, D))


def _readout(h, batch2, charge2, params):
    wc, bc = params['charge']
    w1, b1 = params['pred1']
    w2, b2 = params['pred2']
    w3, b3 = params['pred3']

    def body(h_ref, batch_ref, charge_ref, wc_ref, bc_ref, w1_ref, b1_ref,
             w2_ref, b2_ref, w3_ref, b3_ref, o_ref):
        gids = lax.broadcasted_iota(jnp.int32, (1, G), 1)
        onehot = (batch_ref[...] == gids).astype(_F32)
        dn = (((0,), (0,)), ((), ()))
        sums = lax.dot_general(onehot, h_ref[:, 0:D], dn,
                               preferred_element_type=_F32)
        ones = jnp.ones((N, 1), _F32)
        counts = lax.dot_general(onehot, ones, dn,
                                 preferred_element_type=_F32)
        gmean = sums / jnp.maximum(counts, 1.0)
        cf = charge_ref[...] * wc_ref[...] + bc_ref[...]
        gg = jnp.concatenate([gmean, cf], axis=1)
        o = _sp(jnp.dot(gg, w1_ref[...], preferred_element_type=_F32)
                + b1_ref[...])
        o = _sp(jnp.dot(o, w2_ref[...], preferred_element_type=_F32)
                + b2_ref[...])
        o_ref[...] = (jnp.dot(o, w3_ref[...], preferred_element_type=_F32)
                      + b3_ref[...])

    out = pl.pallas_call(
        body,
        grid=(1,),
        in_specs=[pl.BlockSpec((N, 2 * D), lambda i: (0, 0)),
                  pl.BlockSpec((N, 1), lambda i: (0, 0)),
                  pl.BlockSpec((G, 1), lambda i: (0, 0)),
                  pl.BlockSpec((1, 16), lambda i: (0, 0)),
                  pl.BlockSpec((1, 16), lambda i: (0, 0)),
                  pl.BlockSpec((D + 16, H), lambda i: (0, 0)),
                  pl.BlockSpec((1, H), lambda i: (0, 0)),
                  pl.BlockSpec((H, H), lambda i: (0, 0)),
                  pl.BlockSpec((1, H), lambda i: (0, 0)),
                  pl.BlockSpec((H, 1), lambda i: (0, 0)),
                  pl.BlockSpec((1, 1), lambda i: (0, 0))],
        out_specs=pl.BlockSpec((G, 1), lambda i: (0, 0)),
        out_shape=jax.ShapeDtypeStruct((G, 1), _F32),
    )(h, batch2, charge2, wc, bc.reshape(1, 16), w1, b1.reshape(1, H),
      w2, b2.reshape(1, H), w3, b3.reshape(1, 1))
    return out.reshape(G)


def kernel(x, edge_index, edge_attr, batch, charge, params):
    row3 = edge_index[0].reshape(NW, NCH, CH)
    col3 = edge_index[1].reshape(NW, NCH, CH)

    wa = jnp.pad(params['atom'][0], ((0, 0), (0, D)))
    ba = jnp.pad(params['atom'][1], (0, D))
    h2 = _embed(x, wa, ba, NB)
    e = _embed(edge_attr, params['bond'][0], params['bond'][1], EB)

    for cp in params['convs']:
        t1, t2 = _node_tables(h2, cp)
        r, c = _gather2(t1, t2, row3, col3)
        e, msg = _edge_mlp(r, c, e, cp)
        parts = _scatter_add(msg, col3)
        h2 = _bn_residual(parts, h2, cp['bn_g'], cp['bn_b'])

    return _readout(h2, batch.reshape(N, 1), charge.reshape(G, 1), params)
